# baseline (speedup 1.0000x reference)
import functools

import jax
import jax.numpy as jnp
from jax.experimental import pallas as pl
from jax.experimental.pallas import tpu as pltpu

HID = 64
DSTATE = 16


def _s6_body(xs_ref, delta_ref, bm_ref, cm_ref, a_ref, d_ref, out_ref, h_ref):
    @pl.when(pl.program_id(0) == 0)
    def _init():
        h_ref[...] = jnp.zeros_like(h_ref)

    A = a_ref[...]
    D = d_ref[...]
    bt = xs_ref.shape[0]

    def step(t, h):
        x_t = xs_ref[t, :]
        d_t = delta_ref[t, :]
        b_t = bm_ref[t, :]
        c_t = cm_ref[t, :]
        dA = jnp.exp(d_t[:, None] * A)
        u = (d_t * x_t)[:, None] * b_t[None, :]
        h = dA * h + u
        y = (h * c_t[None, :]).sum(-1) + D[0] * x_t
        out_ref[t, :] = y
        return h

    h_ref[...] = jax.lax.fori_loop(0, bt, step, h_ref[...])


def _s6_pallas(x, p):
    T = x.shape[0]
    delta = jax.nn.softplus(x @ p["W_delta"] + p["b_delta"])
    A = -jnp.exp(p["A_log"])
    Bm = x @ p["W_B"]
    Cm = x @ p["W_C"]
    BT = 1000
    grid = (T // BT,)
    ys = pl.pallas_call(
        _s6_body,
        grid=grid,
        in_specs=[
            pl.BlockSpec((BT, HID), lambda i: (i, 0)),
            pl.BlockSpec((BT, HID), lambda i: (i, 0)),
            pl.BlockSpec((BT, DSTATE), lambda i: (i, 0)),
            pl.BlockSpec((BT, DSTATE), lambda i: (i, 0)),
            pl.BlockSpec((HID, DSTATE), lambda i: (0, 0)),
            pl.BlockSpec((1, HID), lambda i: (0, 0)),
        ],
        out_specs=pl.BlockSpec((BT, HID), lambda i: (i, 0)),
        out_shape=jax.ShapeDtypeStruct((T, HID), jnp.float32),
        scratch_shapes=[pltpu.VMEM((HID, DSTATE), jnp.float32)],
    )(x, delta, Bm, Cm, A, p["D"].reshape(1, HID))
    return ys


def _mlp2(x, p):
    return jax.nn.silu(x @ p["W1"] + p["b1"]) @ p["W2"] + p["b2"]


def _ln(x, p):
    m = x.mean(-1, keepdims=True)
    v = ((x - m) ** 2).mean(-1, keepdims=True)
    return (x - m) / jnp.sqrt(v + 1e-5) * p["g"] + p["b"]


def _gatv2(s, src, dst, edge_attr, p):
    n = s.shape[0]
    x_l = s @ p["Wl"]
    x_r = s @ p["Wr"]
    e = edge_attr @ p["We"]
    m = jax.nn.leaky_relu(x_l[src] + x_r[dst] + e, 0.2)
    logits = m @ p["att"]
    lmax = jax.ops.segment_max(logits, dst, num_segments=n)
    lmax = jnp.where(jnp.isfinite(lmax), lmax, 0.0)
    ex = jnp.exp(logits - lmax[dst])
    den = jax.ops.segment_sum(ex, dst, num_segments=n)
    alpha = ex / (den[dst] + 1e-16)
    out = jax.ops.segment_sum(alpha[:, None] * x_l[src], dst, num_segments=n)
    return out + p["bias"]


def _encoder(x, edge_index, pos, p):
    s = jnp.clip(x @ p["Ws"] + p["bs"], -10.0, 10.0)
    src, dst = edge_index[0], edge_index[1]
    diff = pos[src] - pos[dst]
    dist = jnp.sqrt(jnp.sum(diff ** 2, axis=-1, keepdims=True) + 1e-12)
    for cp in p["convs"]:
        s_out = _gatv2(s, src, dst, dist, cp)
        s = jax.nn.relu(jnp.clip(s + s_out, -20.0, 20.0))
    v = jnp.zeros((s.shape[0], 16, 3), jnp.float32)
    return s, v


def kernel(t, x_L, pos_L, edge_index_L, x_P, pos_P, edge_index_P,
           pocket_center, params):
    t_emb = _mlp2(t.reshape(-1, 1), params["time_mlp"])
    s_L, _ = _encoder(x_L, edge_index_L, pos_L, params["lig"])
    s_P, _ = _encoder(x_P, edge_index_P, pos_P, params["prot"])
    n = s_L.shape[0]
    t_nodes = jnp.broadcast_to(t_emb, (n, HID))
    center = jnp.broadcast_to(pocket_center, (n, 3))
    s_L = s_L + t_nodes
    dist_c = jnp.linalg.norm(pos_L - center, axis=-1, keepdims=True)
    s_L = s_L + _mlp2(dist_c, params["center_proj"])
    s_L = _ln(s_L + _s6_pallas(s_L, params["s6"]), params["ln"])
    s_glob = jnp.clip(s_L.mean(0, keepdims=True), -50.0, 50.0)
    mu = jnp.clip(s_glob @ params["vib_mean"]["W"] + params["vib_mean"]["b"],
                  -10.0, 10.0)
    logvar = jnp.clip(
        s_glob @ params["vib_logvar"]["W"] + params["vib_logvar"]["b"],
        -10.0, 10.0)
    kl = -0.5 * jnp.sum(1.0 + logvar - mu ** 2 - jnp.exp(logvar), axis=-1)
    return (s_L, s_P, mu, logvar, kl)


# speedup vs baseline: 4.5188x; 4.5188x over previous
import functools

import jax
import jax.numpy as jnp
from jax import lax
from jax.experimental import pallas as pl
from jax.experimental.pallas import tpu as pltpu
from jax.experimental.pallas import tpu_sc as plsc

HID = 64
DSTATE = 16
N = 50000
E = 799999
NC = 2
NS = 16
LN = 16
K = 512
E_PAD = 802816
EP128 = E_PAD // 128
NHALF = 25000
NHP = 25024
RPT = NHP // NS
HH = 32
ACCH = 48

_MESH = plsc.VectorSubcoreMesh(core_axis_name="c", subcore_axis_name="s",
                               num_cores=NC, num_subcores=NS)
_SC_PARAMS = pltpu.CompilerParams(use_tc_tiling_on_sc=False,
                                  needs_layout_passes=False)


def _iota16():
    return lax.iota(jnp.int32, LN)


def _full16(v, dtype=jnp.int32):
    return jnp.full((LN,), v, dtype)


def _sc_dist_kernel(pos_hbm, src_hbm, dst_hbm, out_hbm,
                    srcv, dstv, ps, pd, dv, s1, s2, s3, s4):
    wid = lax.axis_index("s") * NC + lax.axis_index("c")
    nb = E_PAD // (NC * NS) // K
    base128 = wid * (nb * (K // 128))

    def block(i, _):
        off128 = base128 + i * (K // 128)
        c1 = pltpu.async_copy(src_hbm.at[pl.ds(off128, 4)], srcv, s1)
        c2 = pltpu.async_copy(dst_hbm.at[pl.ds(off128, 4)], dstv, s2)
        c1.wait()
        c2.wait()
        sems = (s1, s2, s3, s4)
        cps = []
        for q in range(4):
            cps.append(pltpu.async_copy(
                pos_hbm.at[srcv.at[q]], ps.at[pl.ds(q * 128, 128)], sems[q]))
        for cp in cps:
            cp.wait()
        cps = []
        for q in range(4):
            cps.append(pltpu.async_copy(
                pos_hbm.at[dstv.at[q]], pd.at[pl.ds(q * 128, 128)], sems[q]))
        for cp in cps:
            cp.wait()

        def grp(g, _):
            eids = g * LN + _iota16()
            acc = jnp.zeros((LN,), jnp.float32)
            for c in range(3):
                ci = _full16(c)
                d = (plsc.load_gather(ps, [eids, ci])
                     - plsc.load_gather(pd, [eids, ci]))
                acc = acc + d * d
            x = acc + 1e-12
            xi = plsc.bitcast(x, jnp.int32)
            y = plsc.bitcast(jnp.int32(0x5F3759DF) - (xi >> 1), jnp.float32)
            for _ in range(3):
                y = y * (1.5 - 0.5 * x * y * y)
            dist = x * y
            plsc.store_scatter(dv, [eids >> 7, eids & 127], dist)
            return 0

        lax.fori_loop(0, K // LN, grp, 0)
        pltpu.sync_copy(dv, out_hbm.at[pl.ds(off128, 4)])
        return 0

    lax.fori_loop(0, nb, block, 0)


def _sc_dist(pos4, src2d, dst2d):
    k = functools.partial(
        pl.kernel,
        out_type=jax.ShapeDtypeStruct((EP128, 128), jnp.float32),
        mesh=_MESH,
        compiler_params=_SC_PARAMS,
        scratch_types=[
            pltpu.VMEM((4, 128), jnp.int32),
            pltpu.VMEM((4, 128), jnp.int32),
            pltpu.VMEM((K, 8), jnp.float32),
            pltpu.VMEM((K, 8), jnp.float32),
            pltpu.VMEM((4, 128), jnp.float32),
            pltpu.SemaphoreType.DMA,
            pltpu.SemaphoreType.DMA,
            pltpu.SemaphoreType.DMA,
            pltpu.SemaphoreType.DMA,
        ],
    )(_sc_dist_kernel)
    return k(pos4, src2d, dst2d)


def _sc_logits_kernel(xll_hbm, xlh_hbm, xrl_hbm, xrh_hbm, dist_hbm,
                      src_hbm, dst_hbm, we_hbm, att_hbm, lo_hbm, mx_hbm,
                      srcv, dstv, distv, xll, xlh, xrl, xrh,
                      lgv, mxv, wec, attv,
                      s1, s2, s3, s4):
    wid = lax.axis_index("s") * NC + lax.axis_index("c")
    nb = E_PAD // (NC * NS) // K
    base128 = wid * (nb * (K // 128))
    pltpu.sync_copy(we_hbm, wec)
    pltpu.sync_copy(att_hbm, attv)

    def block(i, gm):
        off128 = base128 + i * (K // 128)
        c1 = pltpu.async_copy(src_hbm.at[pl.ds(off128, 4)], srcv, s1)
        c2 = pltpu.async_copy(dst_hbm.at[pl.ds(off128, 4)], dstv, s2)
        c3 = pltpu.async_copy(dist_hbm.at[pl.ds(off128, 4)], distv, s3)
        c1.wait()
        c2.wait()
        c3.wait()
        sems = (s1, s2, s3, s4)
        for idxr, dsts in ((srcv, (xll_hbm, xll)), (srcv, (xlh_hbm, xlh)),
                           (dstv, (xrl_hbm, xrl)), (dstv, (xrh_hbm, xrh))):
            hbm, buf = dsts
            cps = []
            for q in range(4):
                cps.append(pltpu.async_copy(
                    hbm.at[idxr.at[q]], buf.at[pl.ds(q * 128, 128)], sems[q]))
            for cp in cps:
                cp.wait()

        def grp(g, gmi):
            eids = g * LN + _iota16()
            rowv = eids >> 7
            colv = eids & 127
            dv = plsc.load_gather(distv, [rowv, colv])
            acc = jnp.zeros((LN,), jnp.float32)
            for h in range(HID):
                hv = _full16(h)
                hv2 = _full16(h % HH)
                if h < HH:
                    vl = plsc.load_gather(xll, [eids, hv2])
                    vr = plsc.load_gather(xrl, [eids, hv2])
                else:
                    vl = plsc.load_gather(xlh, [eids, hv2])
                    vr = plsc.load_gather(xrh, [eids, hv2])
                z = vl + vr + dv * plsc.load_gather(wec, [hv])
                m = jnp.maximum(z, 0.2 * z)
                acc = acc + m * plsc.load_gather(attv, [hv])
            geid = off128 * 128 + eids
            lg = jnp.where(geid < E, acc, -1e30)
            plsc.store_scatter(lgv, [rowv, colv], lg)
            return jnp.maximum(gmi, lg)

        gm = lax.fori_loop(0, K // LN, grp, gm)
        pltpu.sync_copy(lgv, lo_hbm.at[pl.ds(off128, 4)])
        return gm

    gm = lax.fori_loop(0, nb, block, jnp.full((LN,), -3e38, jnp.float32))
    mxv[...] = gm
    pltpu.sync_copy(mxv, mx_hbm.at[wid])


def _sc_logits(xll, xlh, xrl, xrh, dist2d, src2d, dst2d, wecol, att):
    k = functools.partial(
        pl.kernel,
        out_type=(jax.ShapeDtypeStruct((EP128, 128), jnp.float32),
                  jax.ShapeDtypeStruct((NC * NS, LN), jnp.float32)),
        mesh=_MESH,
        compiler_params=_SC_PARAMS,
        scratch_types=[
            pltpu.VMEM((4, 128), jnp.int32),
            pltpu.VMEM((4, 128), jnp.int32),
            pltpu.VMEM((4, 128), jnp.float32),
            pltpu.VMEM((K, HH), jnp.float32),
            pltpu.VMEM((K, HH), jnp.float32),
            pltpu.VMEM((K, HH), jnp.float32),
            pltpu.VMEM((K, HH), jnp.float32),
            pltpu.VMEM((4, 128), jnp.float32),
            pltpu.VMEM((LN,), jnp.float32),
            pltpu.VMEM((HID,), jnp.float32),
            pltpu.VMEM((HID,), jnp.float32),
            pltpu.SemaphoreType.DMA,
            pltpu.SemaphoreType.DMA,
            pltpu.SemaphoreType.DMA,
            pltpu.SemaphoreType.DMA,
        ],
    )(_sc_logits_kernel)
    return k(xll, xlh, xrl, xrh, dist2d, src2d, dst2d, wecol, att)


def _sc_agg_kernel(xl_hbm, lg_hbm, src_hbm, dst_hbm, mx_hbm, out_hbm,
                   shared, srcv, dstv, lgv, xlr, staged, idxv, exb, mxv,
                   s1, s2, s3, s4):
    cid = lax.axis_index("c")
    sid = lax.axis_index("s")
    ept = E_PAD // NS
    nb = ept // K
    sbase128 = sid * (nb * (K // 128))
    nbase = cid * NHALF

    pltpu.sync_copy(mx_hbm, mxv)
    gmv = jnp.full((LN,), -3e38, jnp.float32)
    for w in range(NC * NS):
        gmv = jnp.maximum(gmv, plsc.load_gather(mxv, [_full16(w), _iota16()]))
    gs = jnp.max(gmv)
    gbc = jnp.full((LN,), gs, jnp.float32)

    zero16 = jnp.zeros((LN,), jnp.float32)

    def zrow(e, _):
        ev = jnp.full((LN,), e, jnp.int32)
        for c in range(ACCH // LN):
            plsc.store_scatter(staged, [ev, c * LN + _iota16()], zero16)
        return 0

    lax.fori_loop(0, K, zrow, 0)
    rbase = sid * RPT
    for j in range(RPT // K):
        pltpu.sync_copy(staged, shared.at[pl.ds(rbase + j * K, K)])
    rem = RPT - (RPT // K) * K
    if rem:
        pltpu.sync_copy(staged.at[pl.ds(0, rem)],
                        shared.at[pl.ds(rbase + (RPT // K) * K, rem)])
    plsc.subcore_barrier()

    def block(i, _):
        off128 = sbase128 + i * (K // 128)
        c1 = pltpu.async_copy(src_hbm.at[pl.ds(off128, 4)], srcv, s1)
        c2 = pltpu.async_copy(dst_hbm.at[pl.ds(off128, 4)], dstv, s2)
        c3 = pltpu.async_copy(lg_hbm.at[pl.ds(off128, 4)], lgv, s3)
        c1.wait()
        c2.wait()
        c3.wait()
        sems = (s1, s2, s3, s4)
        cps = []
        for q in range(4):
            cps.append(pltpu.async_copy(
                xl_hbm.at[srcv.at[q]], xlr.at[pl.ds(q * 128, 128)], sems[q]))
        for cp in cps:
            cp.wait()

        def grp(g, _):
            eids = g * LN + _iota16()
            rowv = eids >> 7
            colv = eids & 127
            lg = plsc.load_gather(lgv, [rowv, colv])
            ex = jnp.exp(lg - gbc)
            d16 = plsc.load_gather(dstv, [rowv, colv])
            rel = d16 - nbase
            inr = (rel >= 0) & (rel < NHALF)
            ex = jnp.where(inr, ex, 0.0)
            idx16 = jnp.where(inr, rel, 0)
            plsc.store_scatter(idxv, [rowv, colv], idx16)
            plsc.store_scatter(exb, [eids], ex)
            plsc.store_scatter(staged, [eids, _full16(HH)], ex)
            return 0

        lax.fori_loop(0, K // LN, grp, 0)

        def edge(e, _):
            ev = jnp.full((LN,), e, jnp.int32)
            exbc = plsc.load_gather(exb, [ev])
            for c in range(HH // LN):
                ci = c * LN + _iota16()
                v = plsc.load_gather(xlr, [ev, ci])
                plsc.store_scatter(staged, [ev, ci], v * exbc)
            return 0

        lax.fori_loop(0, K, edge, 0)
        for q in range(4):
            pltpu.sync_copy(staged.at[pl.ds(q * 128, 128)],
                            shared.at[idxv.at[q]], add=True)
        return 0

    lax.fori_loop(0, nb, block, 0)
    plsc.subcore_barrier()
    pltpu.sync_copy(shared.at[pl.ds(sid * RPT, RPT)],
                    out_hbm.at[pl.ds(cid * NHP + sid * RPT, RPT)])


def _sc_aggregate(xl_half, lg2d, src2d, dst2d, wmax):
    k = functools.partial(
        pl.kernel,
        out_type=jax.ShapeDtypeStruct((NC * NHP, ACCH), jnp.float32),
        mesh=_MESH,
        compiler_params=_SC_PARAMS,
        scratch_types=[
            pltpu.VMEM_SHARED((NHP, ACCH), jnp.float32),
            pltpu.VMEM((4, 128), jnp.int32),
            pltpu.VMEM((4, 128), jnp.int32),
            pltpu.VMEM((4, 128), jnp.float32),
            pltpu.VMEM((K, HH), jnp.float32),
            pltpu.VMEM((K, ACCH), jnp.float32),
            pltpu.VMEM((4, 128), jnp.int32),
            pltpu.VMEM((K,), jnp.float32),
            pltpu.VMEM((NC * NS, LN), jnp.float32),
            pltpu.SemaphoreType.DMA,
            pltpu.SemaphoreType.DMA,
            pltpu.SemaphoreType.DMA,
            pltpu.SemaphoreType.DMA,
        ],
    )(_sc_agg_kernel)
    return k(xl_half, lg2d, src2d, dst2d, wmax)


_BN = 2000


def _entry_body(x_ref, ws_ref, bs_ref, wl_ref, wr_ref,
                s_ref, xll_ref, xlh_ref, xrl_ref, xrh_ref):
    s0 = jnp.clip(
        jnp.dot(x_ref[...], ws_ref[...], preferred_element_type=jnp.float32)
        + bs_ref[...], -10.0, 10.0)
    s_ref[...] = s0
    xl = jnp.dot(s0, wl_ref[...], preferred_element_type=jnp.float32)
    xr = jnp.dot(s0, wr_ref[...], preferred_element_type=jnp.float32)
    xll_ref[...] = xl[:, :HH]
    xlh_ref[...] = xl[:, HH:]
    xrl_ref[...] = xr[:, :HH]
    xrh_ref[...] = xr[:, HH:]


def _tc_entry(x, ws, bs, wl, wr):
    nin = x.shape[1]
    f64 = jax.ShapeDtypeStruct((N, HID), jnp.float32)
    f32h = jax.ShapeDtypeStruct((N, HH), jnp.float32)
    return pl.pallas_call(
        _entry_body,
        grid=(N // _BN,),
        in_specs=[
            pl.BlockSpec((_BN, nin), lambda i: (i, 0)),
            pl.BlockSpec((nin, HID), lambda i: (0, 0)),
            pl.BlockSpec((1, HID), lambda i: (0, 0)),
            pl.BlockSpec((HID, HID), lambda i: (0, 0)),
            pl.BlockSpec((HID, HID), lambda i: (0, 0)),
        ],
        out_specs=[pl.BlockSpec((_BN, HID), lambda i: (i, 0))]
        + [pl.BlockSpec((_BN, HH), lambda i: (i, 0))] * 4,
        out_shape=[f64, f32h, f32h, f32h, f32h],
    )(x, ws, bs.reshape(1, HID), wl, wr)


def _merge_acc(al_ref, ah_ref):
    al = al_ref[...]
    ah = ah_ref[...]
    den = al[:, HH:HH + 1] + 1e-16
    return jnp.concatenate([al[:, :HH], ah[:, :HH]], axis=1) / den


def _mid_body(al_ref, ah_ref, s_ref, b_ref, wl_ref, wr_ref,
              s2_ref, xll_ref, xlh_ref, xrl_ref, xrh_ref):
    out = _merge_acc(al_ref, ah_ref)
    s2 = jax.nn.relu(jnp.clip(s_ref[...] + out + b_ref[...], -20.0, 20.0))
    s2_ref[...] = s2
    xl = jnp.dot(s2, wl_ref[...], preferred_element_type=jnp.float32)
    xr = jnp.dot(s2, wr_ref[...], preferred_element_type=jnp.float32)
    xll_ref[...] = xl[:, :HH]
    xlh_ref[...] = xl[:, HH:]
    xrl_ref[...] = xr[:, :HH]
    xrh_ref[...] = xr[:, HH:]


def _tc_update_mid(acc_lo, acc_hi, s, bias, wl_next, wr_next):
    f64 = jax.ShapeDtypeStruct((N, HID), jnp.float32)
    f32h = jax.ShapeDtypeStruct((N, HH), jnp.float32)
    return pl.pallas_call(
        _mid_body,
        grid=(N // _BN,),
        in_specs=[
            pl.BlockSpec((_BN, ACCH), lambda i: (i, 0)),
            pl.BlockSpec((_BN, ACCH), lambda i: (i, 0)),
            pl.BlockSpec((_BN, HID), lambda i: (i, 0)),
            pl.BlockSpec((1, HID), lambda i: (0, 0)),
            pl.BlockSpec((HID, HID), lambda i: (0, 0)),
            pl.BlockSpec((HID, HID), lambda i: (0, 0)),
        ],
        out_specs=[pl.BlockSpec((_BN, HID), lambda i: (i, 0))]
        + [pl.BlockSpec((_BN, HH), lambda i: (i, 0))] * 4,
        out_shape=[f64, f32h, f32h, f32h, f32h],
    )(acc_lo, acc_hi, s, bias.reshape(1, HID), wl_next, wr_next)


def _last_body(al_ref, ah_ref, s_ref, b_ref, s2_ref):
    out = _merge_acc(al_ref, ah_ref)
    s2_ref[...] = jax.nn.relu(
        jnp.clip(s_ref[...] + out + b_ref[...], -20.0, 20.0))


def _tc_update_last(acc_lo, acc_hi, s, bias):
    return pl.pallas_call(
        _last_body,
        grid=(N // _BN,),
        in_specs=[
            pl.BlockSpec((_BN, ACCH), lambda i: (i, 0)),
            pl.BlockSpec((_BN, ACCH), lambda i: (i, 0)),
            pl.BlockSpec((_BN, HID), lambda i: (i, 0)),
            pl.BlockSpec((1, HID), lambda i: (0, 0)),
        ],
        out_specs=pl.BlockSpec((_BN, HID), lambda i: (i, 0)),
        out_shape=jax.ShapeDtypeStruct((N, HID), jnp.float32),
    )(acc_lo, acc_hi, s, bias.reshape(1, HID))


def _pre_body(s_ref, pos_ref, c_ref, t_ref,
              tw1, tb1, tw2, tb2, cw1, cb1, cw2, cb2,
              wd, bd, wb, wc, dD,
              s_out, delta_out, bm_out, cm_out, dx_out, cdx_out):
    t_emb = (jax.nn.silu(
        jnp.dot(t_ref[...], tw1[...], preferred_element_type=jnp.float32)
        + tb1[...]) @ tw2[...]) + tb2[...]
    d = pos_ref[...] - c_ref[...]
    dist = jnp.sqrt(jnp.sum(d * d, axis=-1, keepdims=True))
    h1 = jax.nn.silu(
        jnp.dot(dist, cw1[...], preferred_element_type=jnp.float32) + cb1[...])
    s = (s_ref[...] + t_emb
         + jnp.dot(h1, cw2[...], preferred_element_type=jnp.float32)
         + cb2[...])
    s_out[...] = s
    delta = jax.nn.softplus(
        jnp.dot(s, wd[...], preferred_element_type=jnp.float32) + bd[...])
    delta_out[...] = delta
    bm_out[...] = jnp.dot(s, wb[...], preferred_element_type=jnp.float32)
    cm_out[...] = jnp.dot(s, wc[...], preferred_element_type=jnp.float32)
    dx_out[...] = delta * s
    cdx_out[...] = dD[...] * s


def _tc_pre_s6(s3, pos, center, t, p_time, p_center, p_s6):
    f64 = jax.ShapeDtypeStruct((N, HID), jnp.float32)
    f16 = jax.ShapeDtypeStruct((N, DSTATE), jnp.float32)
    w64 = pl.BlockSpec((HID, HID), lambda i: (0, 0))
    one64 = pl.BlockSpec((1, HID), lambda i: (0, 0))
    return pl.pallas_call(
        _pre_body,
        grid=(N // _BN,),
        in_specs=[
            pl.BlockSpec((_BN, HID), lambda i: (i, 0)),
            pl.BlockSpec((_BN, 3), lambda i: (i, 0)),
            pl.BlockSpec((1, 3), lambda i: (0, 0)),
            pl.BlockSpec((1, 1), lambda i: (0, 0)),
            pl.BlockSpec((1, HID), lambda i: (0, 0)),
            one64,
            w64,
            one64,
            pl.BlockSpec((1, HID), lambda i: (0, 0)),
            one64,
            w64,
            one64,
            w64,
            one64,
            pl.BlockSpec((HID, DSTATE), lambda i: (0, 0)),
            pl.BlockSpec((HID, DSTATE), lambda i: (0, 0)),
            one64,
        ],
        out_specs=[pl.BlockSpec((_BN, HID), lambda i: (i, 0)),
                   pl.BlockSpec((_BN, HID), lambda i: (i, 0)),
                   pl.BlockSpec((_BN, DSTATE), lambda i: (i, 0)),
                   pl.BlockSpec((_BN, DSTATE), lambda i: (i, 0)),
                   pl.BlockSpec((_BN, HID), lambda i: (i, 0)),
                   pl.BlockSpec((_BN, HID), lambda i: (i, 0))],
        out_shape=[f64, f64, f16, f16, f64, f64],
    )(s3, pos, center, t.reshape(1, 1),
      p_time["W1"], p_time["b1"].reshape(1, HID),
      p_time["W2"], p_time["b2"].reshape(1, HID),
      p_center["W1"], p_center["b1"].reshape(1, HID),
      p_center["W2"], p_center["b2"].reshape(1, HID),
      p_s6["W_delta"], p_s6["b_delta"].reshape(1, HID),
      p_s6["W_B"], p_s6["W_C"], p_s6["D"].reshape(1, HID))


_S6_BT = 1000


def _s6_body(delta_ref, dx_ref, cdx_ref, bm_ref, cm_ref, a_ref,
             out_ref, h_ref):
    @pl.when(pl.program_id(0) == 0)
    def _init():
        h_ref[...] = jnp.zeros_like(h_ref)

    A = a_ref[...]
    bt = delta_ref.shape[0]

    def step(t, h):
        d_t = delta_ref[t, :]
        b_t = bm_ref[t, :]
        c_t = cm_ref[t, :]
        dA = jnp.exp(d_t[:, None] * A)
        u = dx_ref[t, :][:, None] * b_t[None, :]
        h = dA * h + u
        out_ref[t, :] = (h * c_t[None, :]).sum(-1) + cdx_ref[t, :]
        return h

    h_ref[...] = jax.lax.fori_loop(0, bt, step, h_ref[...])


def _s6_scan(delta, dx, cdx, bm, cm, a_log):
    A = -jnp.exp(a_log)
    return pl.pallas_call(
        _s6_body,
        grid=(N // _S6_BT,),
        in_specs=[
            pl.BlockSpec((_S6_BT, HID), lambda i: (i, 0)),
            pl.BlockSpec((_S6_BT, HID), lambda i: (i, 0)),
            pl.BlockSpec((_S6_BT, HID), lambda i: (i, 0)),
            pl.BlockSpec((_S6_BT, DSTATE), lambda i: (i, 0)),
            pl.BlockSpec((_S6_BT, DSTATE), lambda i: (i, 0)),
            pl.BlockSpec((HID, DSTATE), lambda i: (0, 0)),
        ],
        out_specs=pl.BlockSpec((_S6_BT, HID), lambda i: (i, 0)),
        out_shape=jax.ShapeDtypeStruct((N, HID), jnp.float32),
        scratch_shapes=[pltpu.VMEM((HID, DSTATE), jnp.float32)],
    )(delta, dx, cdx, bm, cm, A)


def _post_body(s_ref, ys_ref, g_ref, b_ref, wm_ref, bm_ref, wv_ref, bv_ref,
               sl_ref, mu_ref, lv_ref, kl_ref, sum_ref):
    z = s_ref[...] + ys_ref[...]
    m = z.mean(-1, keepdims=True)
    v = ((z - m) ** 2).mean(-1, keepdims=True)
    sl = (z - m) / jnp.sqrt(v + 1e-5) * g_ref[...] + b_ref[...]
    sl_ref[...] = sl

    @pl.when(pl.program_id(0) == 0)
    def _init():
        sum_ref[...] = jnp.zeros_like(sum_ref)

    sum_ref[...] += jnp.sum(sl, axis=0, keepdims=True)

    @pl.when(pl.program_id(0) == pl.num_programs(0) - 1)
    def _final():
        sg = jnp.clip(sum_ref[...] * (1.0 / N), -50.0, 50.0)
        mu = jnp.clip(
            jnp.dot(sg, wm_ref[...], preferred_element_type=jnp.float32)
            + bm_ref[...], -10.0, 10.0)
        lv = jnp.clip(
            jnp.dot(sg, wv_ref[...], preferred_element_type=jnp.float32)
            + bv_ref[...], -10.0, 10.0)
        mu_ref[...] = mu
        lv_ref[...] = lv
        kl_ref[...] = -0.5 * jnp.sum(
            1.0 + lv - mu ** 2 - jnp.exp(lv), axis=-1, keepdims=True)


def _tc_post(s, ys, p_ln, p_mu, p_lv):
    one64 = pl.BlockSpec((1, HID), lambda i: (0, 0))
    w64 = pl.BlockSpec((HID, HID), lambda i: (0, 0))
    return pl.pallas_call(
        _post_body,
        grid=(N // _BN,),
        in_specs=[
            pl.BlockSpec((_BN, HID), lambda i: (i, 0)),
            pl.BlockSpec((_BN, HID), lambda i: (i, 0)),
            one64, one64, w64, one64, w64, one64,
        ],
        out_specs=[pl.BlockSpec((_BN, HID), lambda i: (i, 0)),
                   one64, one64,
                   pl.BlockSpec((1, 1), lambda i: (0, 0))],
        out_shape=[jax.ShapeDtypeStruct((N, HID), jnp.float32),
                   jax.ShapeDtypeStruct((1, HID), jnp.float32),
                   jax.ShapeDtypeStruct((1, HID), jnp.float32),
                   jax.ShapeDtypeStruct((1, 1), jnp.float32)],
        scratch_shapes=[pltpu.VMEM((1, HID), jnp.float32)],
    )(s, ys, p_ln["g"].reshape(1, HID), p_ln["b"].reshape(1, HID),
      p_mu["W"], p_mu["b"].reshape(1, HID),
      p_lv["W"], p_lv["b"].reshape(1, HID))


def _encoder(x, edge_index, pos, p):
    src = edge_index[0]
    dst = edge_index[1]
    padlen = E_PAD - E
    src2d = jnp.concatenate(
        [src, jnp.zeros((padlen,), jnp.int32)]).reshape(EP128, 128)
    dst2d = jnp.concatenate(
        [dst, jnp.zeros((padlen,), jnp.int32)]).reshape(EP128, 128)
    pos8 = jnp.concatenate(
        [pos, jnp.zeros((pos.shape[0], 5), jnp.float32)], axis=1)
    dist2d = _sc_dist(pos8, src2d, dst2d)

    cp0 = p["convs"][0]
    s, xll, xlh, xrl, xrh = _tc_entry(x, p["Ws"], p["bs"],
                                      cp0["Wl"], cp0["Wr"])
    for li, cp in enumerate(p["convs"]):
        lg2d, wmax = _sc_logits(xll, xlh, xrl, xrh, dist2d, src2d, dst2d,
                                cp["We"].reshape(HID), cp["att"])
        a_lo = _sc_aggregate(xll, lg2d, src2d, dst2d, wmax)
        a_hi = _sc_aggregate(xlh, lg2d, src2d, dst2d, wmax)
        acc_lo = jnp.concatenate([a_lo[:NHALF], a_lo[NHP:NHP + NHALF]], axis=0)
        acc_hi = jnp.concatenate([a_hi[:NHALF], a_hi[NHP:NHP + NHALF]], axis=0)
        if li < 2:
            nxt = p["convs"][li + 1]
            s, xll, xlh, xrl, xrh = _tc_update_mid(
                acc_lo, acc_hi, s, cp["bias"], nxt["Wl"], nxt["Wr"])
        else:
            s = _tc_update_last(acc_lo, acc_hi, s, cp["bias"])
    return s


def kernel(t, x_L, pos_L, edge_index_L, x_P, pos_P, edge_index_P,
           pocket_center, params):
    s3_L = _encoder(x_L, edge_index_L, pos_L, params["lig"])
    s_P = _encoder(x_P, edge_index_P, pos_P, params["prot"])

    s, delta, bm, cm, dx, cdx = _tc_pre_s6(
        s3_L, pos_L, pocket_center, t,
        params["time_mlp"], params["center_proj"], params["s6"])
    ys = _s6_scan(delta, dx, cdx, bm, cm, params["s6"]["A_log"])
    s_L, mu, logvar, kl = _tc_post(
        s, ys, params["ln"], params["vib_mean"], params["vib_logvar"])
    return (s_L, s_P, mu, logvar, kl.reshape(1))


# speedup vs baseline: 4.8140x; 1.0653x over previous
import functools

import jax
import jax.numpy as jnp
from jax import lax
from jax.experimental import pallas as pl
from jax.experimental.pallas import tpu as pltpu
from jax.experimental.pallas import tpu_sc as plsc

HID = 64
DSTATE = 16
N = 50000
E = 799999
NC = 2
NS = 16
LN = 16
K = 512
E_PAD = 802816
EP128 = E_PAD // 128
NHALF = 25000
NHP = 25024
RPT = NHP // NS
HH = 32
ACCH = 48

_MESH = plsc.VectorSubcoreMesh(core_axis_name="c", subcore_axis_name="s",
                               num_cores=NC, num_subcores=NS)
_SC_PARAMS = pltpu.CompilerParams(use_tc_tiling_on_sc=False,
                                  needs_layout_passes=False)


def _iota16():
    return lax.iota(jnp.int32, LN)


def _full16(v, dtype=jnp.int32):
    return jnp.full((LN,), v, dtype)


def _sc_dist_kernel(pos_hbm, src_hbm, dst_hbm, out_hbm,
                    srcv, dstv, ps, pd, dv, s1, s2, s3, s4):
    wid = lax.axis_index("s") * NC + lax.axis_index("c")
    nb = E_PAD // (NC * NS) // K
    base128 = wid * (nb * (K // 128))

    def block(i, _):
        off128 = base128 + i * (K // 128)
        c1 = pltpu.async_copy(src_hbm.at[pl.ds(off128, 4)], srcv, s1)
        c2 = pltpu.async_copy(dst_hbm.at[pl.ds(off128, 4)], dstv, s2)
        c1.wait()
        c2.wait()
        sems = (s1, s2, s3, s4)
        cps = []
        for q in range(4):
            cps.append(pltpu.async_copy(
                pos_hbm.at[srcv.at[q]], ps.at[pl.ds(q * 128, 128)], sems[q]))
        for cp in cps:
            cp.wait()
        cps = []
        for q in range(4):
            cps.append(pltpu.async_copy(
                pos_hbm.at[dstv.at[q]], pd.at[pl.ds(q * 128, 128)], sems[q]))
        for cp in cps:
            cp.wait()

        def grp(g, _):
            eids = g * LN + _iota16()
            acc = jnp.zeros((LN,), jnp.float32)
            for c in range(3):
                ci = _full16(c)
                d = (plsc.load_gather(ps, [eids, ci])
                     - plsc.load_gather(pd, [eids, ci]))
                acc = acc + d * d
            x = acc + 1e-12
            xi = plsc.bitcast(x, jnp.int32)
            y = plsc.bitcast(jnp.int32(0x5F3759DF) - (xi >> 1), jnp.float32)
            for _ in range(3):
                y = y * (1.5 - 0.5 * x * y * y)
            dist = x * y
            plsc.store_scatter(dv, [eids >> 7, eids & 127], dist)
            return 0

        lax.fori_loop(0, K // LN, grp, 0)
        pltpu.sync_copy(dv, out_hbm.at[pl.ds(off128, 4)])
        return 0

    lax.fori_loop(0, nb, block, 0)


def _sc_dist(pos4, src2d, dst2d):
    k = functools.partial(
        pl.kernel,
        out_type=jax.ShapeDtypeStruct((EP128, 128), jnp.float32),
        mesh=_MESH,
        compiler_params=_SC_PARAMS,
        scratch_types=[
            pltpu.VMEM((4, 128), jnp.int32),
            pltpu.VMEM((4, 128), jnp.int32),
            pltpu.VMEM((K, 8), jnp.float32),
            pltpu.VMEM((K, 8), jnp.float32),
            pltpu.VMEM((4, 128), jnp.float32),
            pltpu.SemaphoreType.DMA,
            pltpu.SemaphoreType.DMA,
            pltpu.SemaphoreType.DMA,
            pltpu.SemaphoreType.DMA,
        ],
    )(_sc_dist_kernel)
    return k(pos4, src2d, dst2d)


KL = 256
KL128 = KL // 128
NBL = E_PAD // (NC * NS) // KL


def _sc_logits_kernel(xll_hbm, xlh_hbm, xrl_hbm, xrh_hbm, dist_hbm,
                      src_hbm, dst_hbm, we_hbm, att_hbm, lo_hbm, mx_hbm,
                      srcv0, srcv1, dstv0, dstv1, distv0, distv1,
                      xll0, xll1, xlh0, xlh1, xrl0, xrl1, xrh0, xrh1,
                      lgv, mxv, wec, attv,
                      si0, si1, sg0, sg1):
    wid = lax.axis_index("s") * NC + lax.axis_index("c")
    base128 = wid * (NBL * KL128)
    pltpu.sync_copy(we_hbm, wec)
    pltpu.sync_copy(att_hbm, attv)

    srcv = (srcv0, srcv1)
    dstv = (dstv0, dstv1)
    distv = (distv0, distv1)
    xll = (xll0, xll1)
    xlh = (xlh0, xlh1)
    xrl = (xrl0, xrl1)
    xrh = (xrh0, xrh1)
    sidx = (si0, si1)
    sgat = (sg0, sg1)

    def fire_idx(i, b):
        off128 = base128 + i * KL128
        pltpu.async_copy(src_hbm.at[pl.ds(off128, KL128)], srcv[b], sidx[b])
        pltpu.async_copy(dst_hbm.at[pl.ds(off128, KL128)], dstv[b], sidx[b])
        pltpu.async_copy(dist_hbm.at[pl.ds(off128, KL128)], distv[b], sidx[b])

    def wait_idx(b):
        pltpu.make_async_copy(src_hbm.at[pl.ds(0, KL128)], srcv[b],
                              sidx[b]).wait()
        pltpu.make_async_copy(dst_hbm.at[pl.ds(0, KL128)], dstv[b],
                              sidx[b]).wait()
        pltpu.make_async_copy(dist_hbm.at[pl.ds(0, KL128)], distv[b],
                              sidx[b]).wait()

    def fire_gath(b):
        for idxr, hbm, buf in ((srcv[b], xll_hbm, xll[b]),
                               (srcv[b], xlh_hbm, xlh[b]),
                               (dstv[b], xrl_hbm, xrl[b]),
                               (dstv[b], xrh_hbm, xrh[b])):
            for q in range(KL128):
                pltpu.async_copy(hbm.at[idxr.at[q]],
                                 buf.at[pl.ds(q * 128, 128)], sgat[b])

    def wait_gath(b):
        for buf in (xll[b], xlh[b], xrl[b], xrh[b]):
            for q in range(KL128):
                pltpu.make_async_copy(
                    xll_hbm.at[pl.ds(0, 128)],
                    buf.at[pl.ds(q * 128, 128)], sgat[b]).wait()

    def compute(i, b, gm):
        off128 = base128 + i * KL128

        def grp(g, gmi):
            eids = g * LN + _iota16()
            rowv = eids >> 7
            colv = eids & 127
            dv = plsc.load_gather(distv[b], [rowv, colv])
            acc = jnp.zeros((LN,), jnp.float32)
            for h in range(HID):
                hv = _full16(h)
                hv2 = _full16(h % HH)
                if h < HH:
                    vl = plsc.load_gather(xll[b], [eids, hv2])
                    vr = plsc.load_gather(xrl[b], [eids, hv2])
                else:
                    vl = plsc.load_gather(xlh[b], [eids, hv2])
                    vr = plsc.load_gather(xrh[b], [eids, hv2])
                z = vl + vr + dv * plsc.load_gather(wec, [hv])
                m = jnp.maximum(z, 0.2 * z)
                acc = acc + m * plsc.load_gather(attv, [hv])
            geid = off128 * 128 + eids
            lg = jnp.where(geid < E, acc, -1e30)
            plsc.store_scatter(lgv, [rowv, colv], lg)
            return jnp.maximum(gmi, lg)

        gm = lax.fori_loop(0, KL // LN, grp, gm)
        pltpu.sync_copy(lgv, lo_hbm.at[pl.ds(off128, KL128)])
        return gm

    fire_idx(0, 0)
    wait_idx(0)
    fire_gath(0)
    fire_idx(1, 1)

    def pair(p, gm):
        i = 2 * p
        wait_gath(0)
        wait_idx(1)
        fire_gath(1)
        gm = compute(i, 0, gm)
        fire_idx(i + 2, 0)
        wait_gath(1)
        wait_idx(0)
        fire_gath(0)
        gm = compute(i + 1, 1, gm)
        fire_idx(i + 3, 1)
        return gm

    gm = lax.fori_loop(0, NBL // 2 - 1, pair,
                       jnp.full((LN,), -3e38, jnp.float32))
    wait_gath(0)
    wait_idx(1)
    fire_gath(1)
    gm = compute(NBL - 2, 0, gm)
    wait_gath(1)
    gm = compute(NBL - 1, 1, gm)
    mxv[...] = gm
    pltpu.sync_copy(mxv, mx_hbm.at[wid])


def _sc_logits(xll, xlh, xrl, xrh, dist2d, src2d, dst2d, wecol, att):
    i2 = pltpu.VMEM((KL128, 128), jnp.int32)
    f2 = pltpu.VMEM((KL128, 128), jnp.float32)
    gb = pltpu.VMEM((KL, HH), jnp.float32)
    k = functools.partial(
        pl.kernel,
        out_type=(jax.ShapeDtypeStruct((EP128, 128), jnp.float32),
                  jax.ShapeDtypeStruct((NC * NS, LN), jnp.float32)),
        mesh=_MESH,
        compiler_params=_SC_PARAMS,
        scratch_types=[
            i2, i2, i2, i2, f2, f2,
            gb, gb, gb, gb, gb, gb, gb, gb,
            f2,
            pltpu.VMEM((LN,), jnp.float32),
            pltpu.VMEM((HID,), jnp.float32),
            pltpu.VMEM((HID,), jnp.float32),
            pltpu.SemaphoreType.DMA,
            pltpu.SemaphoreType.DMA,
            pltpu.SemaphoreType.DMA,
            pltpu.SemaphoreType.DMA,
        ],
    )(_sc_logits_kernel)
    return k(xll, xlh, xrl, xrh, dist2d, src2d, dst2d, wecol, att)


def _sc_agg_kernel(xl_hbm, lg_hbm, src_hbm, dst_hbm, mx_hbm, out_hbm,
                   shared, srcv0, srcv1, dstv0, dstv1, lgv0, lgv1,
                   xlr0, xlr1, staged, idxv, exb, mxv,
                   si0, si1, sg0, sg1):
    cid = lax.axis_index("c")
    sid = lax.axis_index("s")
    nb = E_PAD // NS // KL
    sbase128 = sid * (nb * KL128)
    nbase = cid * NHALF

    pltpu.sync_copy(mx_hbm, mxv)
    gmv = jnp.full((LN,), -3e38, jnp.float32)
    for w in range(NC * NS):
        gmv = jnp.maximum(gmv, plsc.load_gather(mxv, [_full16(w), _iota16()]))
    gs = jnp.max(gmv)
    gbc = jnp.full((LN,), gs, jnp.float32)

    zero16 = jnp.zeros((LN,), jnp.float32)

    def zrow(e, _):
        ev = jnp.full((LN,), e, jnp.int32)
        for c in range(ACCH // LN):
            plsc.store_scatter(staged, [ev, c * LN + _iota16()], zero16)
        return 0

    lax.fori_loop(0, KL, zrow, 0)
    rbase = sid * RPT
    for j in range(RPT // KL):
        pltpu.sync_copy(staged, shared.at[pl.ds(rbase + j * KL, KL)])
    rem = RPT - (RPT // KL) * KL
    if rem:
        pltpu.sync_copy(staged.at[pl.ds(0, rem)],
                        shared.at[pl.ds(rbase + (RPT // KL) * KL, rem)])
    plsc.subcore_barrier()

    srcv = (srcv0, srcv1)
    dstv = (dstv0, dstv1)
    lgv = (lgv0, lgv1)
    xlr = (xlr0, xlr1)
    sidx = (si0, si1)
    sgat = (sg0, sg1)

    def fire_idx(i, b):
        off128 = sbase128 + i * KL128
        pltpu.async_copy(src_hbm.at[pl.ds(off128, KL128)], srcv[b], sidx[b])
        pltpu.async_copy(dst_hbm.at[pl.ds(off128, KL128)], dstv[b], sidx[b])
        pltpu.async_copy(lg_hbm.at[pl.ds(off128, KL128)], lgv[b], sidx[b])

    def wait_idx(b):
        pltpu.make_async_copy(src_hbm.at[pl.ds(0, KL128)], srcv[b],
                              sidx[b]).wait()
        pltpu.make_async_copy(dst_hbm.at[pl.ds(0, KL128)], dstv[b],
                              sidx[b]).wait()
        pltpu.make_async_copy(lg_hbm.at[pl.ds(0, KL128)], lgv[b],
                              sidx[b]).wait()

    def fire_gath(b):
        for q in range(KL128):
            pltpu.async_copy(xl_hbm.at[srcv[b].at[q]],
                             xlr[b].at[pl.ds(q * 128, 128)], sgat[b])

    def wait_gath(b):
        for q in range(KL128):
            pltpu.make_async_copy(xl_hbm.at[pl.ds(0, 128)],
                                  xlr[b].at[pl.ds(q * 128, 128)],
                                  sgat[b]).wait()

    def compute(b):
        def grp(g, _):
            eids = g * LN + _iota16()
            rowv = eids >> 7
            colv = eids & 127
            lg = plsc.load_gather(lgv[b], [rowv, colv])
            ex = jnp.exp(lg - gbc)
            d16 = plsc.load_gather(dstv[b], [rowv, colv])
            rel = d16 - nbase
            inr = (rel >= 0) & (rel < NHALF)
            ex = jnp.where(inr, ex, 0.0)
            idx16 = jnp.where(inr, rel, 0)
            plsc.store_scatter(idxv, [rowv, colv], idx16)
            plsc.store_scatter(exb, [eids], ex)
            plsc.store_scatter(staged, [eids, _full16(HH)], ex)
            return 0

        lax.fori_loop(0, KL // LN, grp, 0)

        def edge(e, _):
            ev = jnp.full((LN,), e, jnp.int32)
            exbc = plsc.load_gather(exb, [ev])
            for c in range(HH // LN):
                ci = c * LN + _iota16()
                v = plsc.load_gather(xlr[b], [ev, ci])
                plsc.store_scatter(staged, [ev, ci], v * exbc)
            return 0

        lax.fori_loop(0, KL, edge, 0)
        for q in range(KL128):
            pltpu.sync_copy(staged.at[pl.ds(q * 128, 128)],
                            shared.at[idxv.at[q]], add=True)

    fire_idx(0, 0)
    wait_idx(0)
    fire_gath(0)
    fire_idx(1, 1)

    def pair(p, _):
        i = 2 * p
        wait_gath(0)
        wait_idx(1)
        fire_gath(1)
        compute(0)
        fire_idx(i + 2, 0)
        wait_gath(1)
        wait_idx(0)
        fire_gath(0)
        compute(1)
        fire_idx(i + 3, 1)
        return 0

    lax.fori_loop(0, nb // 2 - 1, pair, 0)
    wait_gath(0)
    wait_idx(1)
    fire_gath(1)
    compute(0)
    wait_gath(1)
    compute(1)
    plsc.subcore_barrier()
    pltpu.sync_copy(shared.at[pl.ds(sid * RPT, RPT)],
                    out_hbm.at[pl.ds(cid * NHP + sid * RPT, RPT)])


def _sc_aggregate(xl_half, lg2d, src2d, dst2d, wmax):
    i2 = pltpu.VMEM((KL128, 128), jnp.int32)
    f2 = pltpu.VMEM((KL128, 128), jnp.float32)
    k = functools.partial(
        pl.kernel,
        out_type=jax.ShapeDtypeStruct((NC * NHP, ACCH), jnp.float32),
        mesh=_MESH,
        compiler_params=_SC_PARAMS,
        scratch_types=[
            pltpu.VMEM_SHARED((NHP, ACCH), jnp.float32),
            i2, i2, i2, i2, f2, f2,
            pltpu.VMEM((KL, HH), jnp.float32),
            pltpu.VMEM((KL, HH), jnp.float32),
            pltpu.VMEM((KL, ACCH), jnp.float32),
            i2,
            pltpu.VMEM((KL,), jnp.float32),
            pltpu.VMEM((NC * NS, LN), jnp.float32),
            pltpu.SemaphoreType.DMA,
            pltpu.SemaphoreType.DMA,
            pltpu.SemaphoreType.DMA,
            pltpu.SemaphoreType.DMA,
        ],
    )(_sc_agg_kernel)
    return k(xl_half, lg2d, src2d, dst2d, wmax)


_BN = 2000


def _entry_body(x_ref, ws_ref, bs_ref, wl_ref, wr_ref,
                s_ref, xll_ref, xlh_ref, xrl_ref, xrh_ref):
    s0 = jnp.clip(
        jnp.dot(x_ref[...], ws_ref[...], preferred_element_type=jnp.float32)
        + bs_ref[...], -10.0, 10.0)
    s_ref[...] = s0
    xl = jnp.dot(s0, wl_ref[...], preferred_element_type=jnp.float32)
    xr = jnp.dot(s0, wr_ref[...], preferred_element_type=jnp.float32)
    xll_ref[...] = xl[:, :HH]
    xlh_ref[...] = xl[:, HH:]
    xrl_ref[...] = xr[:, :HH]
    xrh_ref[...] = xr[:, HH:]


def _tc_entry(x, ws, bs, wl, wr):
    nin = x.shape[1]
    f64 = jax.ShapeDtypeStruct((N, HID), jnp.float32)
    f32h = jax.ShapeDtypeStruct((N, HH), jnp.float32)
    return pl.pallas_call(
        _entry_body,
        grid=(N // _BN,),
        in_specs=[
            pl.BlockSpec((_BN, nin), lambda i: (i, 0)),
            pl.BlockSpec((nin, HID), lambda i: (0, 0)),
            pl.BlockSpec((1, HID), lambda i: (0, 0)),
            pl.BlockSpec((HID, HID), lambda i: (0, 0)),
            pl.BlockSpec((HID, HID), lambda i: (0, 0)),
        ],
        out_specs=[pl.BlockSpec((_BN, HID), lambda i: (i, 0))]
        + [pl.BlockSpec((_BN, HH), lambda i: (i, 0))] * 4,
        out_shape=[f64, f32h, f32h, f32h, f32h],
    )(x, ws, bs.reshape(1, HID), wl, wr)


def _merge_acc(al_ref, ah_ref):
    al = al_ref[...]
    ah = ah_ref[...]
    den = al[:, HH:HH + 1] + 1e-16
    return jnp.concatenate([al[:, :HH], ah[:, :HH]], axis=1) / den


def _mid_body(al_ref, ah_ref, s_ref, b_ref, wl_ref, wr_ref,
              s2_ref, xll_ref, xlh_ref, xrl_ref, xrh_ref):
    out = _merge_acc(al_ref, ah_ref)
    s2 = jax.nn.relu(jnp.clip(s_ref[...] + out + b_ref[...], -20.0, 20.0))
    s2_ref[...] = s2
    xl = jnp.dot(s2, wl_ref[...], preferred_element_type=jnp.float32)
    xr = jnp.dot(s2, wr_ref[...], preferred_element_type=jnp.float32)
    xll_ref[...] = xl[:, :HH]
    xlh_ref[...] = xl[:, HH:]
    xrl_ref[...] = xr[:, :HH]
    xrh_ref[...] = xr[:, HH:]


def _tc_update_mid(acc_lo, acc_hi, s, bias, wl_next, wr_next):
    f64 = jax.ShapeDtypeStruct((N, HID), jnp.float32)
    f32h = jax.ShapeDtypeStruct((N, HH), jnp.float32)
    return pl.pallas_call(
        _mid_body,
        grid=(N // _BN,),
        in_specs=[
            pl.BlockSpec((_BN, ACCH), lambda i: (i, 0)),
            pl.BlockSpec((_BN, ACCH), lambda i: (i, 0)),
            pl.BlockSpec((_BN, HID), lambda i: (i, 0)),
            pl.BlockSpec((1, HID), lambda i: (0, 0)),
            pl.BlockSpec((HID, HID), lambda i: (0, 0)),
            pl.BlockSpec((HID, HID), lambda i: (0, 0)),
        ],
        out_specs=[pl.BlockSpec((_BN, HID), lambda i: (i, 0))]
        + [pl.BlockSpec((_BN, HH), lambda i: (i, 0))] * 4,
        out_shape=[f64, f32h, f32h, f32h, f32h],
    )(acc_lo, acc_hi, s, bias.reshape(1, HID), wl_next, wr_next)


def _last_body(al_ref, ah_ref, s_ref, b_ref, s2_ref):
    out = _merge_acc(al_ref, ah_ref)
    s2_ref[...] = jax.nn.relu(
        jnp.clip(s_ref[...] + out + b_ref[...], -20.0, 20.0))


def _tc_update_last(acc_lo, acc_hi, s, bias):
    return pl.pallas_call(
        _last_body,
        grid=(N // _BN,),
        in_specs=[
            pl.BlockSpec((_BN, ACCH), lambda i: (i, 0)),
            pl.BlockSpec((_BN, ACCH), lambda i: (i, 0)),
            pl.BlockSpec((_BN, HID), lambda i: (i, 0)),
            pl.BlockSpec((1, HID), lambda i: (0, 0)),
        ],
        out_specs=pl.BlockSpec((_BN, HID), lambda i: (i, 0)),
        out_shape=jax.ShapeDtypeStruct((N, HID), jnp.float32),
    )(acc_lo, acc_hi, s, bias.reshape(1, HID))


def _pre_body(s_ref, pos_ref, c_ref, t_ref,
              tw1, tb1, tw2, tb2, cw1, cb1, cw2, cb2,
              wd, bd, wb, wc, dD,
              s_out, delta_out, bm_out, cm_out, dx_out, cdx_out):
    t_emb = (jax.nn.silu(
        jnp.dot(t_ref[...], tw1[...], preferred_element_type=jnp.float32)
        + tb1[...]) @ tw2[...]) + tb2[...]
    d = pos_ref[...] - c_ref[...]
    dist = jnp.sqrt(jnp.sum(d * d, axis=-1, keepdims=True))
    h1 = jax.nn.silu(
        jnp.dot(dist, cw1[...], preferred_element_type=jnp.float32) + cb1[...])
    s = (s_ref[...] + t_emb
         + jnp.dot(h1, cw2[...], preferred_element_type=jnp.float32)
         + cb2[...])
    s_out[...] = s
    delta = jax.nn.softplus(
        jnp.dot(s, wd[...], preferred_element_type=jnp.float32) + bd[...])
    delta_out[...] = delta
    bm_out[...] = jnp.dot(s, wb[...], preferred_element_type=jnp.float32)
    cm_out[...] = jnp.dot(s, wc[...], preferred_element_type=jnp.float32)
    dx_out[...] = delta * s
    cdx_out[...] = dD[...] * s


def _tc_pre_s6(s3, pos, center, t, p_time, p_center, p_s6):
    f64 = jax.ShapeDtypeStruct((N, HID), jnp.float32)
    f16 = jax.ShapeDtypeStruct((N, DSTATE), jnp.float32)
    w64 = pl.BlockSpec((HID, HID), lambda i: (0, 0))
    one64 = pl.BlockSpec((1, HID), lambda i: (0, 0))
    return pl.pallas_call(
        _pre_body,
        grid=(N // _BN,),
        in_specs=[
            pl.BlockSpec((_BN, HID), lambda i: (i, 0)),
            pl.BlockSpec((_BN, 3), lambda i: (i, 0)),
            pl.BlockSpec((1, 3), lambda i: (0, 0)),
            pl.BlockSpec((1, 1), lambda i: (0, 0)),
            pl.BlockSpec((1, HID), lambda i: (0, 0)),
            one64,
            w64,
            one64,
            pl.BlockSpec((1, HID), lambda i: (0, 0)),
            one64,
            w64,
            one64,
            w64,
            one64,
            pl.BlockSpec((HID, DSTATE), lambda i: (0, 0)),
            pl.BlockSpec((HID, DSTATE), lambda i: (0, 0)),
            one64,
        ],
        out_specs=[pl.BlockSpec((_BN, HID), lambda i: (i, 0)),
                   pl.BlockSpec((_BN, HID), lambda i: (i, 0)),
                   pl.BlockSpec((_BN, DSTATE), lambda i: (i, 0)),
                   pl.BlockSpec((_BN, DSTATE), lambda i: (i, 0)),
                   pl.BlockSpec((_BN, HID), lambda i: (i, 0)),
                   pl.BlockSpec((_BN, HID), lambda i: (i, 0))],
        out_shape=[f64, f64, f16, f16, f64, f64],
    )(s3, pos, center, t.reshape(1, 1),
      p_time["W1"], p_time["b1"].reshape(1, HID),
      p_time["W2"], p_time["b2"].reshape(1, HID),
      p_center["W1"], p_center["b1"].reshape(1, HID),
      p_center["W2"], p_center["b2"].reshape(1, HID),
      p_s6["W_delta"], p_s6["b_delta"].reshape(1, HID),
      p_s6["W_B"], p_s6["W_C"], p_s6["D"].reshape(1, HID))


_S6_BT = 1000


def _s6_body(delta_ref, dx_ref, cdx_ref, bm_ref, cm_ref, a_ref,
             out_ref, h_ref):
    @pl.when(pl.program_id(0) == 0)
    def _init():
        h_ref[...] = jnp.zeros_like(h_ref)

    A = a_ref[...]
    bt = delta_ref.shape[0]

    def step(t, h):
        d_t = delta_ref[t, :]
        b_t = bm_ref[t, :]
        c_t = cm_ref[t, :]
        dA = jnp.exp(d_t[:, None] * A)
        u = dx_ref[t, :][:, None] * b_t[None, :]
        h = dA * h + u
        out_ref[t, :] = (h * c_t[None, :]).sum(-1) + cdx_ref[t, :]
        return h

    h_ref[...] = jax.lax.fori_loop(0, bt, step, h_ref[...])


def _s6_scan(delta, dx, cdx, bm, cm, a_log):
    A = -jnp.exp(a_log)
    return pl.pallas_call(
        _s6_body,
        grid=(N // _S6_BT,),
        in_specs=[
            pl.BlockSpec((_S6_BT, HID), lambda i: (i, 0)),
            pl.BlockSpec((_S6_BT, HID), lambda i: (i, 0)),
            pl.BlockSpec((_S6_BT, HID), lambda i: (i, 0)),
            pl.BlockSpec((_S6_BT, DSTATE), lambda i: (i, 0)),
            pl.BlockSpec((_S6_BT, DSTATE), lambda i: (i, 0)),
            pl.BlockSpec((HID, DSTATE), lambda i: (0, 0)),
        ],
        out_specs=pl.BlockSpec((_S6_BT, HID), lambda i: (i, 0)),
        out_shape=jax.ShapeDtypeStruct((N, HID), jnp.float32),
        scratch_shapes=[pltpu.VMEM((HID, DSTATE), jnp.float32)],
    )(delta, dx, cdx, bm, cm, A)


def _post_body(s_ref, ys_ref, g_ref, b_ref, wm_ref, bm_ref, wv_ref, bv_ref,
               sl_ref, mu_ref, lv_ref, kl_ref, sum_ref):
    z = s_ref[...] + ys_ref[...]
    m = z.mean(-1, keepdims=True)
    v = ((z - m) ** 2).mean(-1, keepdims=True)
    sl = (z - m) / jnp.sqrt(v + 1e-5) * g_ref[...] + b_ref[...]
    sl_ref[...] = sl

    @pl.when(pl.program_id(0) == 0)
    def _init():
        sum_ref[...] = jnp.zeros_like(sum_ref)

    sum_ref[...] += jnp.sum(sl, axis=0, keepdims=True)

    @pl.when(pl.program_id(0) == pl.num_programs(0) - 1)
    def _final():
        sg = jnp.clip(sum_ref[...] * (1.0 / N), -50.0, 50.0)
        mu = jnp.clip(
            jnp.dot(sg, wm_ref[...], preferred_element_type=jnp.float32)
            + bm_ref[...], -10.0, 10.0)
        lv = jnp.clip(
            jnp.dot(sg, wv_ref[...], preferred_element_type=jnp.float32)
            + bv_ref[...], -10.0, 10.0)
        mu_ref[...] = mu
        lv_ref[...] = lv
        kl_ref[...] = -0.5 * jnp.sum(
            1.0 + lv - mu ** 2 - jnp.exp(lv), axis=-1, keepdims=True)


def _tc_post(s, ys, p_ln, p_mu, p_lv):
    one64 = pl.BlockSpec((1, HID), lambda i: (0, 0))
    w64 = pl.BlockSpec((HID, HID), lambda i: (0, 0))
    return pl.pallas_call(
        _post_body,
        grid=(N // _BN,),
        in_specs=[
            pl.BlockSpec((_BN, HID), lambda i: (i, 0)),
            pl.BlockSpec((_BN, HID), lambda i: (i, 0)),
            one64, one64, w64, one64, w64, one64,
        ],
        out_specs=[pl.BlockSpec((_BN, HID), lambda i: (i, 0)),
                   one64, one64,
                   pl.BlockSpec((1, 1), lambda i: (0, 0))],
        out_shape=[jax.ShapeDtypeStruct((N, HID), jnp.float32),
                   jax.ShapeDtypeStruct((1, HID), jnp.float32),
                   jax.ShapeDtypeStruct((1, HID), jnp.float32),
                   jax.ShapeDtypeStruct((1, 1), jnp.float32)],
        scratch_shapes=[pltpu.VMEM((1, HID), jnp.float32)],
    )(s, ys, p_ln["g"].reshape(1, HID), p_ln["b"].reshape(1, HID),
      p_mu["W"], p_mu["b"].reshape(1, HID),
      p_lv["W"], p_lv["b"].reshape(1, HID))


def _encoder(x, edge_index, pos, p):
    src = edge_index[0]
    dst = edge_index[1]
    padlen = E_PAD - E
    src2d = jnp.concatenate(
        [src, jnp.zeros((padlen,), jnp.int32)]).reshape(EP128, 128)
    dst2d = jnp.concatenate(
        [dst, jnp.zeros((padlen,), jnp.int32)]).reshape(EP128, 128)
    pos8 = jnp.concatenate(
        [pos, jnp.zeros((pos.shape[0], 5), jnp.float32)], axis=1)
    dist2d = _sc_dist(pos8, src2d, dst2d)

    cp0 = p["convs"][0]
    s, xll, xlh, xrl, xrh = _tc_entry(x, p["Ws"], p["bs"],
                                      cp0["Wl"], cp0["Wr"])
    for li, cp in enumerate(p["convs"]):
        lg2d, wmax = _sc_logits(xll, xlh, xrl, xrh, dist2d, src2d, dst2d,
                                cp["We"].reshape(HID), cp["att"])
        a_lo = _sc_aggregate(xll, lg2d, src2d, dst2d, wmax)
        a_hi = _sc_aggregate(xlh, lg2d, src2d, dst2d, wmax)
        acc_lo = jnp.concatenate([a_lo[:NHALF], a_lo[NHP:NHP + NHALF]], axis=0)
        acc_hi = jnp.concatenate([a_hi[:NHALF], a_hi[NHP:NHP + NHALF]], axis=0)
        if li < 2:
            nxt = p["convs"][li + 1]
            s, xll, xlh, xrl, xrh = _tc_update_mid(
                acc_lo, acc_hi, s, cp["bias"], nxt["Wl"], nxt["Wr"])
        else:
            s = _tc_update_last(acc_lo, acc_hi, s, cp["bias"])
    return s


def kernel(t, x_L, pos_L, edge_index_L, x_P, pos_P, edge_index_P,
           pocket_center, params):
    s3_L = _encoder(x_L, edge_index_L, pos_L, params["lig"])
    s_P = _encoder(x_P, edge_index_P, pos_P, params["prot"])

    s, delta, bm, cm, dx, cdx = _tc_pre_s6(
        s3_L, pos_L, pocket_center, t,
        params["time_mlp"], params["center_proj"], params["s6"])
    ys = _s6_scan(delta, dx, cdx, bm, cm, params["s6"]["A_log"])
    s_L, mu, logvar, kl = _tc_post(
        s, ys, params["ln"], params["vib_mean"], params["vib_logvar"])
    return (s_L, s_P, mu, logvar, kl.reshape(1))
